# Initial kernel scaffold; baseline (speedup 1.0000x reference)
import functools

import jax
import jax.numpy as jnp
from jax import lax
from jax.experimental import pallas as pl
from jax.experimental.pallas import tpu as pltpu
from jax.experimental.pallas import tpu_sc as plsc

_B = 128
_N = 4096
_T = 2.0
_NC = 2
_NS = 16
_NW = _NC * _NS
_BPW = _B // _NW
_NA = 9
_VPR = _N // 16
_NEG = jnp.float32(-jnp.inf)


@functools.partial(
    pl.kernel,
    mesh=plsc.VectorSubcoreMesh(core_axis_name="c", subcore_axis_name="s"),
    out_type=[
        jax.ShapeDtypeStruct((_B * 16,), jnp.float32),
        jax.ShapeDtypeStruct((_NW * 16,), jnp.float32),
    ],
    scratch_types=[
        pltpu.VMEM((_N,), jnp.float32),
        pltpu.VMEM((_N,), jnp.float32),
        pltpu.VMEM((_B,), jnp.int32),
        pltpu.VMEM((_BPW * 16,), jnp.float32),
        pltpu.VMEM((16,), jnp.float32),
        pltpu.SemaphoreType.DMA,
        pltpu.SemaphoreType.DMA,
    ],
)
def _sc_stage(o1, o2, o3, o4, o5, o6, o7, o8, mim, tgt_hbm,
              marg_out, part_out,
              buf0, buf1, tgt_v, marg_v, pm_v, sem0, sem1):
    refs = [o1, o2, o3, o4, o5, o6, o7, o8, mim]
    wid = lax.axis_index("c") * _NS + lax.axis_index("s")
    b0 = wid * _BPW

    pltpu.sync_copy(tgt_hbm, tgt_v)
    zeros16 = jnp.zeros((16,), jnp.float32)
    for bi in range(_BPW):
        marg_v[pl.ds(bi * 16, 16)] = zeros16

    rows = [(a, bi) for bi in range(_BPW) for a in range(_NA)]
    bufs = [buf0, buf1]
    sems = [sem0, sem1]

    def start(k):
        a, bi = rows[k]
        return pltpu.async_copy(refs[a].at[b0 + bi], bufs[k % 2], sems[k % 2])

    pending = start(0)
    pm = _NEG
    neg_vec = jnp.full((16,), _NEG)
    lane = lax.iota(jnp.int32, 16)

    for k, (a, bi) in enumerate(rows):
        buf = bufs[k % 2]
        cur = pending
        if k + 1 < len(rows):
            pending = start(k + 1)
        cur.wait()

        def mbody(i, accs, buf=buf):
            a0, a1, a2, a3 = accs
            base = i * 256
            vs = [buf[pl.ds(base + u * 16, 16)] for u in range(16)]
            for u in range(0, 16, 4):
                a0 = jnp.maximum(a0, vs[u])
                a1 = jnp.maximum(a1, vs[u + 1])
                a2 = jnp.maximum(a2, vs[u + 2])
                a3 = jnp.maximum(a3, vs[u + 3])
            return a0, a1, a2, a3

        a0, a1, a2, a3 = lax.fori_loop(
            0, _VPR // 16, mbody, (neg_vec, neg_vec, neg_vec, neg_vec))
        m = jnp.max(jnp.maximum(jnp.maximum(a0, a1), jnp.maximum(a2, a3)))

        if a < 8:
            pm = jnp.maximum(pm, m)

        t = tgt_v[b0 + bi]
        v_t = buf[t]
        idx = bi * 16 + a

        @pl.when(v_t >= m)
        def _(buf=buf, t=t, m=m, idx=idx):
            def sbody(j, acc, buf=buf, t=t):
                v = buf[pl.ds(j * 16, 16)]
                pos = j * 16 + lane
                return jnp.maximum(acc, jnp.where(pos == t, _NEG, v))

            acc2 = lax.fori_loop(0, _VPR, sbody, neg_vec)
            marg_v[idx] = m - jnp.max(acc2)

    pm_v[...] = jnp.full((16,), pm)
    pltpu.sync_copy(marg_v, marg_out.at[pl.ds(wid * (_BPW * 16), _BPW * 16)])
    pltpu.sync_copy(pm_v, part_out.at[pl.ds(wid * 16, 16)])


def _tc_body(marg_ref, part_ref, thr_ref, mx_ref):
    x = marg_ref[...]
    lanes = lax.broadcasted_iota(jnp.int32, (_B, 16), 1)
    valid = lanes < _NA
    logits = x * jnp.float32(1.0 / _T)
    mrow = jnp.max(jnp.where(valid, logits, jnp.float32(-1e30)),
                   axis=1, keepdims=True)
    e = jnp.where(valid, jnp.exp(logits - mrow), jnp.float32(0.0))
    s = jnp.sum(e, axis=1, keepdims=True)
    out = e / s
    thr_ref[...] = out[:, :_NA]
    mx_ref[...] = jnp.full((1, 1), jnp.max(part_ref[...]))


_tc_stage = pl.pallas_call(
    _tc_body,
    out_shape=(
        jax.ShapeDtypeStruct((_B, _NA), jnp.float32),
        jax.ShapeDtypeStruct((1, 1), jnp.float32),
    ),
)


def kernel(outputs1, outputs2, outputs3, outputs4, outputs5, outputs6,
           outputs7, outputs8, mimic, targets, n_test):
    marg_flat, parts = _sc_stage(
        outputs1, outputs2, outputs3, outputs4, outputs5, outputs6,
        outputs7, outputs8, mimic, targets.astype(jnp.int32))
    thr, mx = _tc_stage(marg_flat.reshape(_B, 16), parts.reshape(_NW, 16))
    return mx.reshape(()), thr


# speedup vs baseline: 23.8784x; 23.8784x over previous
import functools

import jax
import jax.numpy as jnp
from jax import lax
from jax.experimental import pallas as pl
from jax.experimental.pallas import tpu as pltpu
from jax.experimental.pallas import tpu_sc as plsc

_B = 128
_N = 4096
_T = 2.0
_NC = 2
_NS = 16
_NW = _NC * _NS
_BPW = _B // _NW
_NA = 9
_VPR = _N // 16
_NEG = float("-inf")


@functools.partial(
    pl.kernel,
    mesh=plsc.VectorSubcoreMesh(core_axis_name="c", subcore_axis_name="s"),
    out_type=[
        jax.ShapeDtypeStruct((_B * 16,), jnp.float32),
        jax.ShapeDtypeStruct((_NW * 16,), jnp.float32),
    ],
    scratch_types=[
        pltpu.VMEM((_N,), jnp.float32),
        pltpu.VMEM((_N,), jnp.float32),
        pltpu.VMEM((_B,), jnp.int32),
        pltpu.VMEM((_BPW * 16,), jnp.float32),
        pltpu.VMEM((16,), jnp.float32),
        pltpu.SemaphoreType.DMA,
        pltpu.SemaphoreType.DMA,
    ],
    compiler_params=pltpu.CompilerParams(needs_layout_passes=False),
)
def _sc_stage(o1, o2, o3, o4, o5, o6, o7, o8, mim, tgt_hbm,
              marg_out, part_out,
              buf0, buf1, tgt_v, marg_v, pm_v, sem0, sem1):
    refs = [o1, o2, o3, o4, o5, o6, o7, o8, mim]
    wid = lax.axis_index("c") * _NS + lax.axis_index("s")
    b0 = wid * _BPW

    pltpu.sync_copy(tgt_hbm, tgt_v)

    rows = [(a, bi) for bi in range(_BPW) for a in range(_NA)]
    bufs = [buf0, buf1]
    sems = [sem0, sem1]

    def start(k):
        a, bi = rows[k]
        return pltpu.async_copy(refs[a].at[b0 + bi], bufs[k % 2], sems[k % 2])

    pending = start(0)
    pm = _NEG
    neg_vec = jnp.full((16,), _NEG)
    lane = lax.iota(jnp.int32, 16)
    marg_vec = jnp.zeros((16,), jnp.float32)

    for k, (a, bi) in enumerate(rows):
        buf = bufs[k % 2]
        cur = pending
        if k + 1 < len(rows):
            pending = start(k + 1)

        t_all = plsc.load_gather(tgt_v, [jnp.full((16,), b0 + bi, jnp.int32)])
        if a == 0:
            marg_vec = jnp.zeros((16,), jnp.float32)

        cur.wait()
        v_t = plsc.load_gather(buf, [t_all])

        def mbody(i, accs, buf=buf, t_all=t_all):
            a0, a1, a2, a3 = accs
            base = i * 256
            acc = [a0, a1, a2, a3]
            for u in range(16):
                v = buf[pl.ds(base + u * 16, 16)]
                pos = (base + u * 16) + lane
                vm = jnp.where(pos == t_all, _NEG, v)
                acc[u % 4] = jnp.maximum(acc[u % 4], vm)
            return tuple(acc)

        a0, a1, a2, a3 = lax.fori_loop(
            0, _VPR // 16, mbody, (neg_vec, neg_vec, neg_vec, neg_vec))
        me_vec = jnp.maximum(jnp.maximum(a0, a1), jnp.maximum(a2, a3))
        me = jnp.max(me_vec)
        m = jnp.max(jnp.maximum(me_vec, v_t))

        if a < 8:
            pm = jnp.maximum(pm, m)

        marg_vec = jnp.where(lane == a, m - me, marg_vec)
        if a == _NA - 1:
            marg_v[pl.ds(bi * 16, 16)] = marg_vec

    pm_v[...] = jnp.full((16,), pm)
    pltpu.sync_copy(marg_v, marg_out.at[pl.ds(wid * (_BPW * 16), _BPW * 16)])
    pltpu.sync_copy(pm_v, part_out.at[pl.ds(wid * 16, 16)])


def _tc_body(marg_ref, part_ref, thr_ref, mx_ref):
    x = marg_ref[...]
    lanes = lax.broadcasted_iota(jnp.int32, (_B, 16), 1)
    valid = lanes < _NA
    logits = x * jnp.float32(1.0 / _T)
    mrow = jnp.max(jnp.where(valid, logits, jnp.float32(-1e30)),
                   axis=1, keepdims=True)
    e = jnp.where(valid, jnp.exp(logits - mrow), jnp.float32(0.0))
    s = jnp.sum(e, axis=1, keepdims=True)
    out = e / s
    thr_ref[...] = out[:, :_NA]
    mx_ref[...] = jnp.full((1, 1), jnp.max(part_ref[...]))


_tc_stage = pl.pallas_call(
    _tc_body,
    out_shape=(
        jax.ShapeDtypeStruct((_B, _NA), jnp.float32),
        jax.ShapeDtypeStruct((1, 1), jnp.float32),
    ),
)


def kernel(outputs1, outputs2, outputs3, outputs4, outputs5, outputs6,
           outputs7, outputs8, mimic, targets, n_test):
    marg_flat, parts = _sc_stage(
        outputs1, outputs2, outputs3, outputs4, outputs5, outputs6,
        outputs7, outputs8, mimic, targets.astype(jnp.int32))
    thr, mx = _tc_stage(marg_flat.reshape(_B, 16), parts.reshape(_NW, 16))
    return mx.reshape(()), thr
